# baseline (speedup 1.0000x reference)
import functools

import numpy as np
import jax
import jax.numpy as jnp
from jax import lax
from jax.experimental import pallas as pl
from jax.experimental.pallas import tpu as pltpu
from jax.experimental.pallas import tpu_sc as plsc

N_LEVELS = 16
N_FEAT = 2
LOG2_HASHMAP = 19
TABLE_SIZE = 2 ** LOG2_HASHMAP
MASK = TABLE_SIZE - 1
BASE_RES = 16
FINEST_RES = 512
N_POINTS = 262144

_LEVEL_RES = [int(np.floor(BASE_RES * np.exp(l * np.log(FINEST_RES / BASE_RES) / (N_LEVELS - 1))))
              for l in range(N_LEVELS)]
_P2 = 2654435761
_P3 = 805459861

NW = 32
PPW = N_POINTS // NW
P = 512
NCHUNK = PPW // P


def _wrap32(v):
    v &= 0xFFFFFFFF
    return v - (1 << 32) if v >= (1 << 31) else v


def _build_consts():
    tbl = np.zeros((N_LEVELS, 16, 16), dtype=np.int32)
    for l, R in enumerate(_LEVEL_RES):
        A = _wrap32(R)
        B = _wrap32(R * _P2)
        C = _wrap32(R * _P3)
        corners = [(dx, dy, dz) for dx in (0, 1) for dy in (0, 1) for dz in (0, 1)]
        for k, (dx, dy, dz) in enumerate(corners):
            tbl[l, k, :] = _wrap32(dx * A + dy * B + dz * C)
        tbl[l, 8, :] = A
        tbl[l, 9, :] = B
        tbl[l, 10, :] = C
        tbl[l, 11, :] = l * TABLE_SIZE
    return tbl


_CONSTS = _build_consts()
_CONSTS_F = np.zeros((N_LEVELS, 1, 16), dtype=np.float32)
for _l, _R in enumerate(_LEVEL_RES):
    _CONSTS_F[_l, 0, :] = np.float32(_R)


def _sc_call(cx, cy, cz, tab, consts, consts_f):
    mesh = plsc.VectorSubcoreMesh(core_axis_name="c", subcore_axis_name="s")

    @functools.partial(
        pl.kernel,
        mesh=mesh,
        out_type=jax.ShapeDtypeStruct((NW * NCHUNK, 2 * N_LEVELS, P), jnp.float32),
        compiler_params=pltpu.CompilerParams(needs_layout_passes=False),
        scratch_types=[
            pltpu.VMEM((PPW,), jnp.float32),
            pltpu.VMEM((PPW,), jnp.float32),
            pltpu.VMEM((PPW,), jnp.float32),
            pltpu.VMEM((N_LEVELS, 16, 16), jnp.int32),
            pltpu.VMEM((N_LEVELS, 1, 16), jnp.float32),
            pltpu.VMEM((8 * P,), jnp.int32),
            pltpu.VMEM((8 * P,), jnp.float32),
            pltpu.VMEM((8 * P,), jnp.int32),
            pltpu.VMEM((8 * P,), jnp.float32),
            pltpu.VMEM((2 * N_LEVELS, P), jnp.float32),
            pltpu.VMEM((4928,), jnp.float32),
            pltpu.VMEM((9280,), jnp.float32),
            pltpu.VMEM((17600,), jnp.float32),
            pltpu.SemaphoreType.DMA,
            pltpu.SemaphoreType.DMA,
        ],
    )
    def body(cx_h, cy_h, cz_h, tab_h, consts_h, constsf_h, out_h,
             cx_v, cy_v, cz_v, consts_v, constsf_v,
             ia_v, ra_v, ib_v, rb_v,
             out_v, g0_v, g1_v, g2_v, sema, semb):
        wid = lax.axis_index("s") * np.int32(2) + lax.axis_index("c")
        base_pt = wid * np.int32(PPW)
        pltpu.sync_copy(cx_h.at[pl.ds(base_pt, PPW)], cx_v)
        pltpu.sync_copy(cy_h.at[pl.ds(base_pt, PPW)], cy_v)
        pltpu.sync_copy(cz_h.at[pl.ds(base_pt, PPW)], cz_v)
        pltpu.sync_copy(consts_h, consts_v)
        pltpu.sync_copy(constsf_h, constsf_v)

        iota = lax.iota(jnp.int32, 16)

        for lvl, grid_v, gsize in ((0, g0_v, 4928), (1, g1_v, 9280),
                                   (2, g2_v, 17600)):
            nrounds = -(-gsize // 4096)
            R = _LEVEL_RES[lvl]
            S = np.int32(R + 1)
            S2 = np.int32((R + 1) * (R + 1))
            A = np.int32(_wrap32(R))
            B = np.int32(_wrap32(R * _P2))
            C = np.int32(_wrap32(R * _P3))
            loff = np.int32(lvl * TABLE_SIZE)
            for r in range(nrounds):
                rbase = np.int32(r * 4096)

                def ggen(i, _, rbase=rbase, S=S, S2=S2, A=A, B=B, C=C,
                         loff=loff):
                    e = iota + (rbase + i * np.int32(16))
                    a = e // S2
                    rem = e - a * S2
                    b = rem // S
                    cc = rem - b * S
                    gi = ((a * A + b * B + cc * C) & np.int32(MASK)) + loff
                    ia_v[pl.ds(i * np.int32(16), 16)] = gi
                    return np.int32(0)

                lax.fori_loop(np.int32(0), np.int32(256), ggen, np.int32(0),
                              unroll=False)
                pltpu.async_copy(tab_h.at[ia_v], ra_v, sema)
                pltpu.make_async_copy(tab_h.at[ia_v], ra_v, sema).wait()

                def gcopy(i, _, rbase=rbase, grid_v=grid_v):
                    sl16 = pl.ds(i * np.int32(16), 16)
                    grid_v[pl.ds(rbase + i * np.int32(16), 16)] = ra_v[sl16]
                    return np.int32(0)

                niter = (min(gsize, (r + 1) * 4096) - r * 4096 + 15) // 16
                lax.fori_loop(np.int32(0), np.int32(niter), gcopy, np.int32(0),
                              unroll=False)

        def chunk_body(c, _):
            def run_p1(lvl, idx_v):
                Af = consts_v[lvl, 8, :]
                Bf = consts_v[lvl, 9, :]
                Cf = consts_v[lvl, 10, :]
                loff = consts_v[lvl, 11, :]
                Rf = constsf_v[lvl, 0, :]

                def p1(i8, _):
                    for ii in range(4):
                        off = c * np.int32(P) + i8 * np.int32(64) + np.int32(ii * 16)
                        sl = pl.ds(off, 16)
                        fxi = (cx_v[sl] * Rf).astype(jnp.int32)
                        fyi = (cy_v[sl] * Rf).astype(jnp.int32)
                        fzi = (cz_v[sl] * Rf).astype(jnp.int32)
                        base = fxi * Af + fyi * Bf + fzi * Cf
                        for k in range(8):
                            h = base if k == 0 else base + consts_v[lvl, np.int32(k), :]
                            csl = pl.ds(np.int32(k * P) + i8 * np.int32(64) + np.int32(ii * 16), 16)
                            idx_v[csl] = (h & np.int32(MASK)) + loff
                    return np.int32(0)

                lax.fori_loop(np.int32(0), np.int32(8), p1, np.int32(0), unroll=False)

            def run_p2(lvl, rows_v):
                Rf = constsf_v[lvl, 0, :]

                def p2(i8, _):
                    for ii in range(4):
                        pbase = i8 * np.int32(64) + np.int32(ii * 16)
                        psl = pl.ds(pbase, 16)
                        sl = pl.ds(c * np.int32(P) + pbase, 16)
                        one = np.float32(1.0)
                        sx = cx_v[sl] * Rf
                        sy = cy_v[sl] * Rf
                        sz = cz_v[sl] * Rf
                        frx = sx - sx.astype(jnp.int32).astype(jnp.float32)
                        fry = sy - sy.astype(jnp.int32).astype(jnp.float32)
                        frz = sz - sz.astype(jnp.int32).astype(jnp.float32)
                        gx = one - frx
                        gy = one - fry
                        gz = one - frz
                        cw = []
                        for k in range(8):
                            w = rows_v[pl.ds(np.int32(k * P) + pbase, 16)]
                            cw.append(plsc.unpack(
                                plsc.bitcast(w, jnp.bfloat16),
                                format=plsc.PackFormat.INTERLEAVED))
                        for f in range(N_FEAT):
                            cv = [cw[k][f] for k in range(8)]
                            c00 = cv[0] * gz + cv[1] * frz
                            c01 = cv[2] * gz + cv[3] * frz
                            c10 = cv[4] * gz + cv[5] * frz
                            c11 = cv[6] * gz + cv[7] * frz
                            c0 = c00 * gy + c01 * fry
                            c1 = c10 * gy + c11 * fry
                            res = c0 * gx + c1 * frx
                            out_v[lvl * np.int32(2) + np.int32(f), psl] = res
                    return np.int32(0)

                lax.fori_loop(np.int32(0), np.int32(8), p2, np.int32(0), unroll=False)

            def fire(idx_v, rows_v, sem):
                pltpu.async_copy(tab_h.at[idx_v], rows_v, sem)

            def drain(idx_v, rows_v, sem):
                pltpu.make_async_copy(tab_h.at[idx_v], rows_v, sem).wait()

            def run_dense(lvl, grid_v):
                R = _LEVEL_RES[lvl]
                S = R + 1
                Rf = np.float32(R)
                GA = [np.int32(dx * S * S + dy * S + dz)
                      for dx in (0, 1) for dy in (0, 1) for dz in (0, 1)]

                def pd(i8, _):
                    for ii in range(4):
                        pbase = i8 * np.int32(64) + np.int32(ii * 16)
                        psl = pl.ds(pbase, 16)
                        sl = pl.ds(c * np.int32(P) + pbase, 16)
                        one = np.float32(1.0)
                        sx = cx_v[sl] * Rf
                        sy = cy_v[sl] * Rf
                        sz = cz_v[sl] * Rf
                        fxi = sx.astype(jnp.int32)
                        fyi = sy.astype(jnp.int32)
                        fzi = sz.astype(jnp.int32)
                        frx = sx - fxi.astype(jnp.float32)
                        fry = sy - fyi.astype(jnp.float32)
                        frz = sz - fzi.astype(jnp.float32)
                        gx = one - frx
                        gy = one - fry
                        gz = one - frz
                        g000 = (fxi * np.int32(S) + fyi) * np.int32(S) + fzi
                        cw = []
                        for k in range(8):
                            w = plsc.load_gather(grid_v, [g000 + GA[k]])
                            cw.append(plsc.unpack(
                                plsc.bitcast(w, jnp.bfloat16),
                                format=plsc.PackFormat.INTERLEAVED))
                        for f in range(N_FEAT):
                            cv = [cw[k][f] for k in range(8)]
                            c00 = cv[0] * gz + cv[1] * frz
                            c01 = cv[2] * gz + cv[3] * frz
                            c10 = cv[4] * gz + cv[5] * frz
                            c11 = cv[6] * gz + cv[7] * frz
                            c0 = c00 * gy + c01 * fry
                            c1 = c10 * gy + c11 * fry
                            res = c0 * gx + c1 * frx
                            out_v[np.int32(lvl * 2 + f), psl] = res
                    return np.int32(0)

                lax.fori_loop(np.int32(0), np.int32(8), pd, np.int32(0),
                              unroll=False)

            run_dense(0, g0_v)
            run_dense(1, g1_v)
            run_dense(2, g2_v)

            def pair_body(m, _):
                la = np.int32(3) + m * np.int32(2)
                run_p1(la, ia_v)
                fire(ia_v, ra_v, sema)

                @pl.when(m > np.int32(0))
                def _():
                    drain(ib_v, rb_v, semb)
                    run_p2(la - np.int32(1), rb_v)

                run_p1(la + np.int32(1), ib_v)
                fire(ib_v, rb_v, semb)
                drain(ia_v, ra_v, sema)
                run_p2(la, ra_v)
                return np.int32(0)

            lax.fori_loop(np.int32(0), np.int32((N_LEVELS - 4) // 2), pair_body,
                          np.int32(0), unroll=False)
            drain(ib_v, rb_v, semb)
            run_p2(np.int32(N_LEVELS - 2), rb_v)
            run_p1(np.int32(N_LEVELS - 1), ia_v)
            fire(ia_v, ra_v, sema)
            drain(ia_v, ra_v, sema)
            run_p2(np.int32(N_LEVELS - 1), ra_v)
            out_off = wid * np.int32(NCHUNK) + c
            pltpu.sync_copy(out_v, out_h.at[out_off])
            return np.int32(0)

        lax.fori_loop(np.int32(0), np.int32(NCHUNK), chunk_body, np.int32(0),
                      unroll=False)

    return body(cx, cy, cz, tab, consts, consts_f)


def _tc_transpose_body(x_ref, o_ref):
    o_ref[...] = jnp.swapaxes(x_ref[...], 1, 2)


def _tc_transpose(x):
    n_chunks = NW * NCHUNK
    blk = 16
    return pl.pallas_call(
        _tc_transpose_body,
        grid=(n_chunks // blk,),
        in_specs=[pl.BlockSpec((blk, 2 * N_LEVELS, P), lambda i: (i, 0, 0))],
        out_specs=pl.BlockSpec((blk, P, 2 * N_LEVELS), lambda i: (i, 0, 0)),
        out_shape=jax.ShapeDtypeStruct((n_chunks, P, 2 * N_LEVELS), jnp.float32),
    )(x)


def kernel(coords, tables):
    with jax.enable_x64(False):
        coords = coords.astype(jnp.float32)
        cx = coords[:, 0]
        cy = coords[:, 1]
        cz = coords[:, 2]
        tab = lax.bitcast_convert_type(
            tables.astype(jnp.bfloat16),
            jnp.float32).reshape(N_LEVELS * TABLE_SIZE)
        consts = jnp.asarray(_CONSTS)
        consts_f = jnp.asarray(_CONSTS_F)
        out = _sc_call(cx, cy, cz, tab, consts, consts_f)
        return _tc_transpose(out).reshape(N_POINTS, 2 * N_LEVELS)


# speedup vs baseline: 1.0250x; 1.0250x over previous
import functools

import numpy as np
import jax
import jax.numpy as jnp
from jax import lax
from jax.experimental import pallas as pl
from jax.experimental.pallas import tpu as pltpu
from jax.experimental.pallas import tpu_sc as plsc

N_LEVELS = 16
N_FEAT = 2
LOG2_HASHMAP = 19
TABLE_SIZE = 2 ** LOG2_HASHMAP
MASK = TABLE_SIZE - 1
BASE_RES = 16
FINEST_RES = 512
N_POINTS = 262144

_LEVEL_RES = [int(np.floor(BASE_RES * np.exp(l * np.log(FINEST_RES / BASE_RES) / (N_LEVELS - 1))))
              for l in range(N_LEVELS)]
_P2 = 2654435761
_P3 = 805459861

NW = 32
PPW = N_POINTS // NW
P = 512
NCHUNK = PPW // P


def _wrap32(v):
    v &= 0xFFFFFFFF
    return v - (1 << 32) if v >= (1 << 31) else v


def _build_consts():
    tbl = np.zeros((N_LEVELS, 16, 16), dtype=np.int32)
    for l, R in enumerate(_LEVEL_RES):
        A = _wrap32(R)
        B = _wrap32(R * _P2)
        C = _wrap32(R * _P3)
        corners = [(dx, dy, dz) for dx in (0, 1) for dy in (0, 1) for dz in (0, 1)]
        for k, (dx, dy, dz) in enumerate(corners):
            tbl[l, k, :] = _wrap32(dx * A + dy * B + dz * C)
        tbl[l, 8, :] = A
        tbl[l, 9, :] = B
        tbl[l, 10, :] = C
        tbl[l, 11, :] = l * TABLE_SIZE
    return tbl


_CONSTS = _build_consts()
_CONSTS_F = np.zeros((N_LEVELS, 1, 16), dtype=np.float32)
for _l, _R in enumerate(_LEVEL_RES):
    _CONSTS_F[_l, 0, :] = np.float32(_R)


def _sc_call(cx, cy, cz, tab, consts, consts_f):
    mesh = plsc.VectorSubcoreMesh(core_axis_name="c", subcore_axis_name="s")

    @functools.partial(
        pl.kernel,
        mesh=mesh,
        out_type=jax.ShapeDtypeStruct((NW * NCHUNK, 2 * N_LEVELS, P), jnp.float32),
        compiler_params=pltpu.CompilerParams(needs_layout_passes=False),
        scratch_types=[
            pltpu.VMEM((PPW,), jnp.float32),
            pltpu.VMEM((PPW,), jnp.float32),
            pltpu.VMEM((PPW,), jnp.float32),
            pltpu.VMEM((N_LEVELS, 16, 16), jnp.int32),
            pltpu.VMEM((N_LEVELS, 1, 16), jnp.float32),
            pltpu.VMEM((8 * P,), jnp.int32),
            pltpu.VMEM((8 * P,), jnp.float32),
            pltpu.VMEM((8 * P,), jnp.int32),
            pltpu.VMEM((8 * P,), jnp.float32),
            pltpu.VMEM((2 * N_LEVELS, P), jnp.float32),
            pltpu.VMEM((8192,), jnp.float32),
            pltpu.VMEM((12288,), jnp.float32),
            pltpu.SemaphoreType.DMA,
            pltpu.SemaphoreType.DMA,
        ],
    )
    def body(cx_h, cy_h, cz_h, tab_h, consts_h, constsf_h, out_h,
             cx_v, cy_v, cz_v, consts_v, constsf_v,
             ia_v, ra_v, ib_v, rb_v,
             out_v, g0_v, g1_v, sema, semb):
        wid = lax.axis_index("s") * np.int32(2) + lax.axis_index("c")
        base_pt = wid * np.int32(PPW)
        pltpu.sync_copy(cx_h.at[pl.ds(base_pt, PPW)], cx_v)
        pltpu.sync_copy(cy_h.at[pl.ds(base_pt, PPW)], cy_v)
        pltpu.sync_copy(cz_h.at[pl.ds(base_pt, PPW)], cz_v)
        pltpu.sync_copy(consts_h, consts_v)
        pltpu.sync_copy(constsf_h, constsf_v)

        iota = lax.iota(jnp.int32, 16)

        for lvl, grid_v, nrounds in ((0, g0_v, 2), (1, g1_v, 3)):
            R = _LEVEL_RES[lvl]
            S = np.int32(R + 1)
            S2 = np.int32((R + 1) * (R + 1))
            A = np.int32(_wrap32(R))
            B = np.int32(_wrap32(R * _P2))
            C = np.int32(_wrap32(R * _P3))
            loff = np.int32(lvl * TABLE_SIZE)
            for r in range(nrounds):
                rbase = np.int32(r * 4096)

                def ggen(i, _, rbase=rbase, S=S, S2=S2, A=A, B=B, C=C,
                         loff=loff):
                    e = iota + (rbase + i * np.int32(16))
                    a = e // S2
                    rem = e - a * S2
                    b = rem // S
                    cc = rem - b * S
                    gi = ((a * A + b * B + cc * C) & np.int32(MASK)) + loff
                    ia_v[pl.ds(i * np.int32(16), 16)] = gi
                    return np.int32(0)

                lax.fori_loop(np.int32(0), np.int32(256), ggen, np.int32(0),
                              unroll=False)
                pltpu.async_copy(tab_h.at[ia_v], ra_v, sema)
                pltpu.make_async_copy(tab_h.at[ia_v], ra_v, sema).wait()

                def gcopy(i, _, rbase=rbase, grid_v=grid_v):
                    sl16 = pl.ds(i * np.int32(16), 16)
                    grid_v[pl.ds(rbase + i * np.int32(16), 16)] = ra_v[sl16]
                    return np.int32(0)

                lax.fori_loop(np.int32(0), np.int32(256), gcopy, np.int32(0),
                              unroll=False)

        def chunk_body(c, _):
            def run_p1(lvl, idx_v):
                Af = consts_v[lvl, 8, :]
                Bf = consts_v[lvl, 9, :]
                Cf = consts_v[lvl, 10, :]
                loff = consts_v[lvl, 11, :]
                Rf = constsf_v[lvl, 0, :]

                def p1(i8, _):
                    for ii in range(4):
                        off = c * np.int32(P) + i8 * np.int32(64) + np.int32(ii * 16)
                        sl = pl.ds(off, 16)
                        fxi = (cx_v[sl] * Rf).astype(jnp.int32)
                        fyi = (cy_v[sl] * Rf).astype(jnp.int32)
                        fzi = (cz_v[sl] * Rf).astype(jnp.int32)
                        base = fxi * Af + fyi * Bf + fzi * Cf
                        for k in range(8):
                            h = base if k == 0 else base + consts_v[lvl, np.int32(k), :]
                            csl = pl.ds(np.int32(k * P) + i8 * np.int32(64) + np.int32(ii * 16), 16)
                            idx_v[csl] = (h & np.int32(MASK)) + loff
                    return np.int32(0)

                lax.fori_loop(np.int32(0), np.int32(8), p1, np.int32(0), unroll=False)

            def run_p2(lvl, rows_v):
                Rf = constsf_v[lvl, 0, :]

                def p2(i8, _):
                    for ii in range(4):
                        pbase = i8 * np.int32(64) + np.int32(ii * 16)
                        psl = pl.ds(pbase, 16)
                        sl = pl.ds(c * np.int32(P) + pbase, 16)
                        one = np.float32(1.0)
                        sx = cx_v[sl] * Rf
                        sy = cy_v[sl] * Rf
                        sz = cz_v[sl] * Rf
                        frx = sx - sx.astype(jnp.int32).astype(jnp.float32)
                        fry = sy - sy.astype(jnp.int32).astype(jnp.float32)
                        frz = sz - sz.astype(jnp.int32).astype(jnp.float32)
                        gx = one - frx
                        gy = one - fry
                        gz = one - frz
                        cw = []
                        for k in range(8):
                            w = rows_v[pl.ds(np.int32(k * P) + pbase, 16)]
                            cw.append(plsc.unpack(
                                plsc.bitcast(w, jnp.bfloat16),
                                format=plsc.PackFormat.INTERLEAVED))
                        for f in range(N_FEAT):
                            cv = [cw[k][f] for k in range(8)]
                            c00 = cv[0] * gz + cv[1] * frz
                            c01 = cv[2] * gz + cv[3] * frz
                            c10 = cv[4] * gz + cv[5] * frz
                            c11 = cv[6] * gz + cv[7] * frz
                            c0 = c00 * gy + c01 * fry
                            c1 = c10 * gy + c11 * fry
                            res = c0 * gx + c1 * frx
                            out_v[lvl * np.int32(2) + np.int32(f), psl] = res
                    return np.int32(0)

                lax.fori_loop(np.int32(0), np.int32(8), p2, np.int32(0), unroll=False)

            def fire(idx_v, rows_v, sem):
                pltpu.async_copy(tab_h.at[idx_v], rows_v, sem)

            def drain(idx_v, rows_v, sem):
                pltpu.make_async_copy(tab_h.at[idx_v], rows_v, sem).wait()

            def run_dense(lvl, grid_v):
                R = _LEVEL_RES[lvl]
                S = R + 1
                Rf = np.float32(R)
                GA = [np.int32(dx * S * S + dy * S + dz)
                      for dx in (0, 1) for dy in (0, 1) for dz in (0, 1)]

                def pd(i8, _):
                    for ii in range(4):
                        pbase = i8 * np.int32(64) + np.int32(ii * 16)
                        psl = pl.ds(pbase, 16)
                        sl = pl.ds(c * np.int32(P) + pbase, 16)
                        one = np.float32(1.0)
                        sx = cx_v[sl] * Rf
                        sy = cy_v[sl] * Rf
                        sz = cz_v[sl] * Rf
                        fxi = sx.astype(jnp.int32)
                        fyi = sy.astype(jnp.int32)
                        fzi = sz.astype(jnp.int32)
                        frx = sx - fxi.astype(jnp.float32)
                        fry = sy - fyi.astype(jnp.float32)
                        frz = sz - fzi.astype(jnp.float32)
                        gx = one - frx
                        gy = one - fry
                        gz = one - frz
                        g000 = (fxi * np.int32(S) + fyi) * np.int32(S) + fzi
                        cw = []
                        for k in range(8):
                            w = plsc.load_gather(grid_v, [g000 + GA[k]])
                            cw.append(plsc.unpack(
                                plsc.bitcast(w, jnp.bfloat16),
                                format=plsc.PackFormat.INTERLEAVED))
                        for f in range(N_FEAT):
                            cv = [cw[k][f] for k in range(8)]
                            c00 = cv[0] * gz + cv[1] * frz
                            c01 = cv[2] * gz + cv[3] * frz
                            c10 = cv[4] * gz + cv[5] * frz
                            c11 = cv[6] * gz + cv[7] * frz
                            c0 = c00 * gy + c01 * fry
                            c1 = c10 * gy + c11 * fry
                            res = c0 * gx + c1 * frx
                            out_v[np.int32(lvl * 2 + f), psl] = res
                    return np.int32(0)

                lax.fori_loop(np.int32(0), np.int32(8), pd, np.int32(0),
                              unroll=False)

            run_dense(0, g0_v)
            run_dense(1, g1_v)

            def pair_body(m, _):
                la = np.int32(2) + m * np.int32(2)
                run_p1(la, ia_v)
                fire(ia_v, ra_v, sema)

                @pl.when(m > np.int32(0))
                def _():
                    drain(ib_v, rb_v, semb)
                    run_p2(la - np.int32(1), rb_v)

                run_p1(la + np.int32(1), ib_v)
                fire(ib_v, rb_v, semb)
                drain(ia_v, ra_v, sema)
                run_p2(la, ra_v)
                return np.int32(0)

            lax.fori_loop(np.int32(0), np.int32((N_LEVELS - 2) // 2), pair_body,
                          np.int32(0), unroll=False)
            drain(ib_v, rb_v, semb)
            run_p2(np.int32(N_LEVELS - 1), rb_v)
            out_off = wid * np.int32(NCHUNK) + c
            pltpu.sync_copy(out_v, out_h.at[out_off])
            return np.int32(0)

        lax.fori_loop(np.int32(0), np.int32(NCHUNK), chunk_body, np.int32(0),
                      unroll=False)

    return body(cx, cy, cz, tab, consts, consts_f)


def _tc_transpose_body(x_ref, o_ref):
    o_ref[...] = jnp.swapaxes(x_ref[...], 1, 2)


def _tc_transpose(x):
    n_chunks = NW * NCHUNK
    blk = 16
    return pl.pallas_call(
        _tc_transpose_body,
        grid=(n_chunks // blk,),
        in_specs=[pl.BlockSpec((blk, 2 * N_LEVELS, P), lambda i: (i, 0, 0))],
        out_specs=pl.BlockSpec((blk, P, 2 * N_LEVELS), lambda i: (i, 0, 0)),
        out_shape=jax.ShapeDtypeStruct((n_chunks, P, 2 * N_LEVELS), jnp.float32),
    )(x)


def kernel(coords, tables):
    with jax.enable_x64(False):
        coords = coords.astype(jnp.float32)
        cx = coords[:, 0]
        cy = coords[:, 1]
        cz = coords[:, 2]
        tab = lax.bitcast_convert_type(
            tables.astype(jnp.bfloat16),
            jnp.float32).reshape(N_LEVELS * TABLE_SIZE)
        consts = jnp.asarray(_CONSTS)
        consts_f = jnp.asarray(_CONSTS_F)
        out = _sc_call(cx, cy, cz, tab, consts, consts_f)
        return _tc_transpose(out).reshape(N_POINTS, 2 * N_LEVELS)
